# Initial kernel scaffold; baseline (speedup 1.0000x reference)
import jax
import jax.numpy as jnp
from jax.experimental import pallas as pl


def kernel(x, idx, device, emb1, emb2, lin1_w, lin1_b, lin2_w, lin2_b, conv_in_w, conv_in_b, mlp_w, mlp_b, gen_w, gen_b):
    raise NotImplementedError("write your pallas kernel here")


# speedup vs baseline: 5.8174x; 5.8174x over previous
import jax
import jax.numpy as jnp
from jax import lax
from jax.experimental import pallas as pl
from jax.experimental.pallas import tpu as pltpu

N = 1024
D = 64
K = 30
ALPHA = 3.0
PROP_ALPHA = 0.05
C1 = 32
T = 48


def _leaky(v):
    return jnp.where(v >= 0, v, 0.01 * v)


def _graph_kernel(emb1_ref, emb2_ref, l1w_ref, l1b_ref, l2w_ref, l2b_ref,
                  anormt_ref, work_ref, mask_ref):
    f32 = jnp.float32
    n1 = jnp.tanh(ALPHA * (
        lax.dot_general(emb1_ref[...], l1w_ref[...], (((1,), (1,)), ((), ())),
                        preferred_element_type=f32) + l1b_ref[...][None, :]))
    n2 = jnp.tanh(ALPHA * (
        lax.dot_general(emb2_ref[...], l2w_ref[...], (((1,), (1,)), ((), ())),
                        preferred_element_type=f32) + l2b_ref[...][None, :]))
    s1 = lax.dot_general(n1, n2, (((1,), (1,)), ((), ())),
                         preferred_element_type=f32)
    s2 = lax.dot_general(n2, n1, (((1,), (1,)), ((), ())),
                         preferred_element_type=f32)
    b = jax.nn.relu(jnp.tanh(ALPHA * (s2 - s1)))

    rowidx = lax.broadcasted_iota(jnp.int32, (N, N), 0)

    work_ref[...] = b
    mask_ref[...] = jnp.zeros((N, N), f32)

    def body(_, carry):
        work = work_ref[...]
        m = jnp.max(work, axis=0, keepdims=True)
        is_max = work == m
        sel_row = jnp.min(jnp.where(is_max, rowidx, N), axis=0, keepdims=True)
        sel = rowidx == sel_row
        mask_ref[...] = mask_ref[...] + sel.astype(f32)
        work_ref[...] = jnp.where(sel, -1.0, work)
        return carry

    lax.fori_loop(0, K, body, 0)
    adjt = jnp.where(mask_ref[...] > 0, b, 0.0)
    att = adjt + (rowidx == lax.broadcasted_iota(jnp.int32, (N, N), 1)
                  ).astype(f32)
    sums = jnp.sum(att, axis=0, keepdims=True)
    anormt_ref[...] = att / sums


def _main_kernel(x_ref, anormt_ref, w_in_ref, b_in_ref, mlp_w_ref, mlp_b_ref,
                 gen_w_ref, gen_b_ref, out_ref, planes_ref):
    f32 = jnp.float32
    xb = x_ref[0]
    w_in = w_in_ref[...].reshape(1, C1, 1)
    b_in = b_in_ref[...].reshape(1, C1, 1)
    h0 = _leaky(xb[:, None, :] * w_in + b_in)
    h0f = h0.reshape(T * C1, N)
    prop = lax.dot_general(h0f, anormt_ref[...], (((1,), (0,)), ((), ())),
                           preferred_element_type=f32)
    h1f = PROP_ALPHA * h0f + (1.0 - PROP_ALPHA) * prop

    mw = mlp_w_ref[...].reshape(C1, 2 * C1)
    mwa, mwb = mw[:, :C1], mw[:, C1:]
    mlp_b = mlp_b_ref[...][:, None]
    gw = gen_w_ref[...].reshape(C1, 9)

    for t in range(T):
        h0_t = h0f[t * C1:(t + 1) * C1, :]
        h1_t = h1f[t * C1:(t + 1) * C1, :]
        lat_t = (lax.dot_general(mwa, h0_t, (((1,), (0,)), ((), ())),
                                 preferred_element_type=f32)
                 + lax.dot_general(mwb, h1_t, (((1,), (0,)), ((), ())),
                                   preferred_element_type=f32)
                 + mlp_b)
        planes_ref[:, t, :] = lax.dot_general(
            gw, lat_t, (((0,), (0,)), ((), ())), preferred_element_type=f32)

    planes = planes_ref[...]
    tix = lax.broadcasted_iota(jnp.int32, (T, N), 0)
    nix = lax.broadcasted_iota(jnp.int32, (T, N), 1)
    acc = jnp.zeros((T, N), f32)
    for kh in range(3):
        for kw in range(3):
            di, dj = kh - 1, kw - 1
            shifted = planes[kh * 3 + kw]
            if dj:
                shifted = jnp.roll(shifted, -dj, axis=0)
            if di:
                shifted = jnp.roll(shifted, -di, axis=1)
            valid = ((tix + dj >= 0) & (tix + dj < T)
                     & (nix + di >= 0) & (nix + di < N))
            acc = acc + jnp.where(valid, shifted, 0.0)
    out_ref[0] = _leaky(acc + gen_b_ref[...])


def kernel(x, idx, device, emb1, emb2, lin1_w, lin1_b, lin2_w, lin2_b,
           conv_in_w, conv_in_b, mlp_w, mlp_b, gen_w, gen_b):
    del idx, device
    f32 = jnp.float32

    anormt = pl.pallas_call(
        _graph_kernel,
        out_shape=jax.ShapeDtypeStruct((N, N), f32),
        scratch_shapes=[pltpu.VMEM((N, N), f32), pltpu.VMEM((N, N), f32)],
    )(emb1, emb2, lin1_w, lin1_b, lin2_w, lin2_b)

    out = pl.pallas_call(
        _main_kernel,
        grid=(x.shape[0],),
        in_specs=[
            pl.BlockSpec((1, T, N), lambda b: (b, 0, 0)),
            pl.BlockSpec((N, N), lambda b: (0, 0)),
            pl.BlockSpec(conv_in_w.shape, lambda b: (0,) * 4),
            pl.BlockSpec(conv_in_b.shape, lambda b: (0,)),
            pl.BlockSpec(mlp_w.shape, lambda b: (0,) * 4),
            pl.BlockSpec(mlp_b.shape, lambda b: (0,)),
            pl.BlockSpec(gen_w.shape, lambda b: (0,) * 4),
            pl.BlockSpec(gen_b.shape, lambda b: (0,)),
        ],
        out_specs=pl.BlockSpec((1, T, N), lambda b: (b, 0, 0)),
        out_shape=jax.ShapeDtypeStruct((x.shape[0], T, N), f32),
        scratch_shapes=[pltpu.VMEM((9, T, N), f32)],
    )(x, anormt, conv_in_w, conv_in_b, mlp_w, mlp_b, gen_w, gen_b)
    return out


# speedup vs baseline: 8.1754x; 1.4053x over previous
import jax
import jax.numpy as jnp
from jax import lax
from jax.experimental import pallas as pl
from jax.experimental.pallas import tpu as pltpu

N = 1024
D = 64
K = 30
ALPHA = 3.0
PROP_ALPHA = 0.05
C1 = 32
T = 48


def _leaky(v):
    return jnp.where(v >= 0, v, 0.01 * v)


def _graph_kernel(emb1_ref, emb2_ref, l1w_ref, l1b_ref, l2w_ref, l2b_ref,
                  anormt_ref, work_ref, mask_ref):
    f32 = jnp.float32
    n1 = jnp.tanh(ALPHA * (
        lax.dot_general(emb1_ref[...], l1w_ref[...], (((1,), (1,)), ((), ())),
                        preferred_element_type=f32) + l1b_ref[...][None, :]))
    n2 = jnp.tanh(ALPHA * (
        lax.dot_general(emb2_ref[...], l2w_ref[...], (((1,), (1,)), ((), ())),
                        preferred_element_type=f32) + l2b_ref[...][None, :]))
    s1 = lax.dot_general(n1, n2, (((1,), (1,)), ((), ())),
                         preferred_element_type=f32)
    s2 = lax.dot_general(n2, n1, (((1,), (1,)), ((), ())),
                         preferred_element_type=f32)
    b = jax.nn.relu(jnp.tanh(ALPHA * (s2 - s1)))

    rowidx = lax.broadcasted_iota(jnp.int32, (N, N), 0)

    work_ref[...] = b
    mask_ref[...] = jnp.zeros((N, N), f32)

    def body(_, carry):
        work = work_ref[...]
        m = jnp.max(work, axis=0, keepdims=True)
        is_max = work == m
        sel_row = jnp.min(jnp.where(is_max, rowidx, N), axis=0, keepdims=True)
        sel = rowidx == sel_row
        mask_ref[...] = mask_ref[...] + sel.astype(f32)
        work_ref[...] = jnp.where(sel, -1.0, work)
        return carry

    lax.fori_loop(0, K, body, 0)
    adjt = jnp.where(mask_ref[...] > 0, b, 0.0)
    att = adjt + (rowidx == lax.broadcasted_iota(jnp.int32, (N, N), 1)
                  ).astype(f32)
    sums = jnp.sum(att, axis=0, keepdims=True)
    anormt_ref[...] = att / sums


def _main_kernel(x_ref, anormt_ref, w_in_ref, b_in_ref, mlp_w_ref, mlp_b_ref,
                 gen_w_ref, gen_b_ref, out_ref, planes_ref):
    f32 = jnp.float32
    xb = x_ref[0]
    w_in = w_in_ref[...].reshape(1, C1, 1)
    b_in = b_in_ref[...].reshape(1, C1, 1)
    h0 = _leaky(xb[:, None, :] * w_in + b_in)
    h0f = h0.reshape(T * C1, N)
    prop = lax.dot_general(h0f.astype(jnp.bfloat16),
                           anormt_ref[...].astype(jnp.bfloat16),
                           (((1,), (0,)), ((), ())),
                           preferred_element_type=f32)
    h1f = PROP_ALPHA * h0f + (1.0 - PROP_ALPHA) * prop

    mw = mlp_w_ref[...].reshape(C1, 2 * C1)
    mwa, mwb = mw[:, :C1], mw[:, C1:]
    mlp_b = mlp_b_ref[...][:, None]
    gw = gen_w_ref[...].reshape(C1, 9)
    wa = lax.dot_general(gw, mwa, (((0,), (0,)), ((), ())),
                         preferred_element_type=f32)
    wb = lax.dot_general(gw, mwb, (((0,), (0,)), ((), ())),
                         preferred_element_type=f32)
    cst = lax.dot_general(gw, mlp_b, (((0,), (0,)), ((), ())),
                          preferred_element_type=f32)

    for t in range(T):
        h0_t = h0f[t * C1:(t + 1) * C1, :]
        h1_t = h1f[t * C1:(t + 1) * C1, :]
        planes_ref[:, t, :] = (
            lax.dot_general(wa, h0_t, (((1,), (0,)), ((), ())),
                            preferred_element_type=f32)
            + lax.dot_general(wb, h1_t, (((1,), (0,)), ((), ())),
                              preferred_element_type=f32)
            + cst)

    planes = planes_ref[...]
    tix = lax.broadcasted_iota(jnp.int32, (T, N), 0)
    nix = lax.broadcasted_iota(jnp.int32, (T, N), 1)
    acc = jnp.zeros((T, N), f32)
    for kh in range(3):
        for kw in range(3):
            di, dj = kh - 1, kw - 1
            shifted = planes[kh * 3 + kw]
            if dj:
                shifted = jnp.roll(shifted, -dj, axis=0)
            if di:
                shifted = jnp.roll(shifted, -di, axis=1)
            valid = ((tix + dj >= 0) & (tix + dj < T)
                     & (nix + di >= 0) & (nix + di < N))
            acc = acc + jnp.where(valid, shifted, 0.0)
    out_ref[0] = _leaky(acc + gen_b_ref[...])


def kernel(x, idx, device, emb1, emb2, lin1_w, lin1_b, lin2_w, lin2_b,
           conv_in_w, conv_in_b, mlp_w, mlp_b, gen_w, gen_b):
    del idx, device
    f32 = jnp.float32

    anormt = pl.pallas_call(
        _graph_kernel,
        out_shape=jax.ShapeDtypeStruct((N, N), f32),
        scratch_shapes=[pltpu.VMEM((N, N), f32), pltpu.VMEM((N, N), f32)],
    )(emb1, emb2, lin1_w, lin1_b, lin2_w, lin2_b)

    out = pl.pallas_call(
        _main_kernel,
        grid=(x.shape[0],),
        in_specs=[
            pl.BlockSpec((1, T, N), lambda b: (b, 0, 0)),
            pl.BlockSpec((N, N), lambda b: (0, 0)),
            pl.BlockSpec(conv_in_w.shape, lambda b: (0,) * 4),
            pl.BlockSpec(conv_in_b.shape, lambda b: (0,)),
            pl.BlockSpec(mlp_w.shape, lambda b: (0,) * 4),
            pl.BlockSpec(mlp_b.shape, lambda b: (0,)),
            pl.BlockSpec(gen_w.shape, lambda b: (0,) * 4),
            pl.BlockSpec(gen_b.shape, lambda b: (0,)),
        ],
        out_specs=pl.BlockSpec((1, T, N), lambda b: (b, 0, 0)),
        out_shape=jax.ShapeDtypeStruct((x.shape[0], T, N), f32),
        scratch_shapes=[pltpu.VMEM((9, T, N), f32)],
    )(x, anormt, conv_in_w, conv_in_b, mlp_w, mlp_b, gen_w, gen_b)
    return out
